# baseline (speedup 1.0000x reference)
import jax
import jax.numpy as jnp
from jax.experimental import pallas as pl
from jax.experimental.pallas import tpu as pltpu

N = 8192
BM = 256
NI = N // BM


def _gcn_kernel(adj_ref, x_ref, w1_ref, b1_ref, w2_ref, b2_ref, w3_ref,
                b3_ref, wr_ref, br_ref, o_ref, u1, u2, v):
    l = pl.program_id(0)
    i = pl.program_id(1)
    a = adj_ref[...]
    row = pl.ds(i * BM, BM)

    @pl.when(jnp.logical_and(l == 0, i == 0))
    def _init():
        u1[...] = jnp.dot(x_ref[...], w1_ref[...],
                          preferred_element_type=jnp.float32)
        o_ref[...] = br_ref[...]

    @pl.when(l == 0)
    def _layer1():
        h = jnp.maximum(
            jnp.dot(a, u1[...], preferred_element_type=jnp.float32)
            + b1_ref[...], 0.0)
        u2[row, :] = jnp.dot(h, w2_ref[...],
                             preferred_element_type=jnp.float32)

    @pl.when(l == 1)
    def _layer2():
        h = jnp.maximum(
            jnp.dot(a, u2[...], preferred_element_type=jnp.float32)
            + b2_ref[...], 0.0)
        v[row, :] = jnp.dot(h, w3_ref[...],
                            preferred_element_type=jnp.float32)

    @pl.when(l == 2)
    def _layer3():
        h = jnp.maximum(
            jnp.dot(a, v[...], preferred_element_type=jnp.float32)
            + b3_ref[...], 0.0)
        o_ref[...] = o_ref[...] + jnp.dot(
            wr_ref[...], h, preferred_element_type=jnp.float32)


def kernel(x, adj, W1, b1, W2, b2, W3, b3, Wr, br):
    out = pl.pallas_call(
        _gcn_kernel,
        grid=(3, NI),
        in_specs=[
            pl.BlockSpec((BM, N), lambda l, i: (i, 0)),
            pl.BlockSpec((N, 128), lambda l, i: (0, 0)),
            pl.BlockSpec((128, 32), lambda l, i: (0, 0)),
            pl.BlockSpec((1, 32), lambda l, i: (0, 0)),
            pl.BlockSpec((32, 16), lambda l, i: (0, 0)),
            pl.BlockSpec((1, 16), lambda l, i: (0, 0)),
            pl.BlockSpec((16, 1), lambda l, i: (0, 0)),
            pl.BlockSpec((1, 1), lambda l, i: (0, 0)),
            pl.BlockSpec((1, BM), lambda l, i: (0, i)),
            pl.BlockSpec((1, 1), lambda l, i: (0, 0)),
        ],
        out_specs=pl.BlockSpec((1, 1), lambda l, i: (0, 0)),
        out_shape=jax.ShapeDtypeStruct((1, 1), jnp.float32),
        scratch_shapes=[
            pltpu.VMEM((N, 32), jnp.float32),
            pltpu.VMEM((N, 16), jnp.float32),
            pltpu.VMEM((N, 1), jnp.float32),
        ],
    )(adj, x, W1, b1.reshape(1, 32), W2, b2.reshape(1, 16), W3,
      b3.reshape(1, 1), Wr, br.reshape(1, 1))
    return out.reshape(1)


# speedup vs baseline: 1.1283x; 1.1283x over previous
import jax
import jax.numpy as jnp
from jax.experimental import pallas as pl
from jax.experimental.pallas import tpu as pltpu

N = 8192
BM1 = 256
NI1 = N // BM1
BM2 = 512
NI2 = N // BM2


def _proj_kernel(x_ref, w1_ref, u1_ref):
    u1_ref[...] = jnp.dot(x_ref[...], w1_ref[...],
                          preferred_element_type=jnp.float32)


def _layer1_kernel(adj_ref, u1_ref, b1_ref, w2_ref, abf_ref, u2_ref):
    a = adj_ref[...].astype(jnp.bfloat16)
    abf_ref[...] = a
    h = jnp.maximum(
        jnp.dot(a, u1_ref[...].astype(jnp.bfloat16),
                preferred_element_type=jnp.float32)
        + b1_ref[...], 0.0)
    u2_ref[...] = jnp.dot(h, w2_ref[...], preferred_element_type=jnp.float32)


def _layer23_kernel(abf_ref, u2_ref, b2_ref, w3_ref, b3_ref, wr_ref, br_ref,
                    o_ref, s_ref):
    l = pl.program_id(0)
    i = pl.program_id(1)
    a = abf_ref[...]
    row = pl.ds(i * BM2, BM2)

    @pl.when(l == 0)
    def _layer2():
        h = jnp.maximum(
            jnp.dot(a, u2_ref[...].astype(jnp.bfloat16),
                    preferred_element_type=jnp.float32)
            + b2_ref[...], 0.0)
        s_ref[row, 0:1] = jnp.dot(h, w3_ref[...],
                                  preferred_element_type=jnp.float32)

    @pl.when(l == 1)
    def _layer3():
        h = jnp.maximum(
            jnp.dot(a, s_ref[:, 0:1].astype(jnp.bfloat16),
                    preferred_element_type=jnp.float32)
            + b3_ref[...], 0.0)
        s_ref[row, 1:2] = h

    @pl.when(jnp.logical_and(l == 1, i == NI2 - 1))
    def _readout():
        o_ref[...] = br_ref[...] + jnp.sum(
            wr_ref[...] * s_ref[:, 1:2].reshape(1, N), keepdims=True)


def kernel(x, adj, W1, b1, W2, b2, W3, b3, Wr, br):
    u1 = pl.pallas_call(
        _proj_kernel,
        out_shape=jax.ShapeDtypeStruct((N, 32), jnp.float32),
    )(x, W1)
    adj_bf, u2 = pl.pallas_call(
        _layer1_kernel,
        grid=(NI1,),
        in_specs=[
            pl.BlockSpec((BM1, N), lambda i: (i, 0)),
            pl.BlockSpec((N, 32), lambda i: (0, 0)),
            pl.BlockSpec((1, 32), lambda i: (0, 0)),
            pl.BlockSpec((32, 16), lambda i: (0, 0)),
        ],
        out_specs=[
            pl.BlockSpec((BM1, N), lambda i: (i, 0)),
            pl.BlockSpec((BM1, 16), lambda i: (i, 0)),
        ],
        out_shape=[
            jax.ShapeDtypeStruct((N, N), jnp.bfloat16),
            jax.ShapeDtypeStruct((N, 16), jnp.float32),
        ],
    )(adj, u1, b1.reshape(1, 32), W2)
    out = pl.pallas_call(
        _layer23_kernel,
        grid=(2, NI2),
        in_specs=[
            pl.BlockSpec((BM2, N), lambda l, i: (i, 0)),
            pl.BlockSpec((N, 16), lambda l, i: (0, 0)),
            pl.BlockSpec((1, 16), lambda l, i: (0, 0)),
            pl.BlockSpec((16, 1), lambda l, i: (0, 0)),
            pl.BlockSpec((1, 1), lambda l, i: (0, 0)),
            pl.BlockSpec((1, N), lambda l, i: (0, 0)),
            pl.BlockSpec((1, 1), lambda l, i: (0, 0)),
        ],
        out_specs=pl.BlockSpec((1, 1), lambda l, i: (0, 0)),
        out_shape=jax.ShapeDtypeStruct((1, 1), jnp.float32),
        scratch_shapes=[
            pltpu.VMEM((N, 128), jnp.float32),
        ],
    )(adj_bf, u2, b2.reshape(1, 16), W3, b3.reshape(1, 1), Wr,
      br.reshape(1, 1))
    return out.reshape(1)


# speedup vs baseline: 1.1637x; 1.0314x over previous
import jax
import jax.numpy as jnp
from jax.experimental import pallas as pl
from jax.experimental.pallas import tpu as pltpu

N = 8192
BM1 = 512
NI1 = N // BM1
BM2 = 1024
NI2 = N // BM2


def _proj_kernel(x_ref, w1_ref, u1_ref):
    u1_ref[...] = jnp.dot(x_ref[...], w1_ref[...],
                          preferred_element_type=jnp.float32)


def _layer1_kernel(adj_ref, u1_ref, b1_ref, w2_ref, abf_ref, u2_ref):
    a = adj_ref[...].astype(jnp.bfloat16)
    abf_ref[...] = a
    h = jnp.maximum(
        jnp.dot(a, u1_ref[...].astype(jnp.bfloat16),
                preferred_element_type=jnp.float32)
        + b1_ref[...], 0.0)
    u2_ref[...] = jnp.dot(h, w2_ref[...], preferred_element_type=jnp.float32)


def _layer23_kernel(abf_ref, u2_ref, b2_ref, w3_ref, b3_ref, wr_ref, br_ref,
                    o_ref, s_ref):
    l = pl.program_id(0)
    i = pl.program_id(1)
    a = abf_ref[...]
    row = pl.ds(i * BM2, BM2)

    @pl.when(l == 0)
    def _layer2():
        h = jnp.maximum(
            jnp.dot(a, u2_ref[...].astype(jnp.bfloat16),
                    preferred_element_type=jnp.float32)
            + b2_ref[...], 0.0)
        s_ref[row, 0:1] = jnp.dot(h, w3_ref[...],
                                  preferred_element_type=jnp.float32)

    @pl.when(l == 1)
    def _layer3():
        h = jnp.maximum(
            jnp.dot(a, s_ref[:, 0:1].astype(jnp.bfloat16),
                    preferred_element_type=jnp.float32)
            + b3_ref[...], 0.0)
        s_ref[row, 1:2] = h

    @pl.when(jnp.logical_and(l == 1, i == NI2 - 1))
    def _readout():
        o_ref[...] = br_ref[...] + jnp.sum(
            wr_ref[...] * s_ref[:, 1:2].reshape(1, N), keepdims=True)


def kernel(x, adj, W1, b1, W2, b2, W3, b3, Wr, br):
    u1 = pl.pallas_call(
        _proj_kernel,
        out_shape=jax.ShapeDtypeStruct((N, 32), jnp.float32),
    )(x, W1)
    adj_bf, u2 = pl.pallas_call(
        _layer1_kernel,
        grid=(NI1,),
        in_specs=[
            pl.BlockSpec((BM1, N), lambda i: (i, 0)),
            pl.BlockSpec((N, 32), lambda i: (0, 0)),
            pl.BlockSpec((1, 32), lambda i: (0, 0)),
            pl.BlockSpec((32, 16), lambda i: (0, 0)),
        ],
        out_specs=[
            pl.BlockSpec((BM1, N), lambda i: (i, 0)),
            pl.BlockSpec((BM1, 16), lambda i: (i, 0)),
        ],
        out_shape=[
            jax.ShapeDtypeStruct((N, N), jnp.bfloat16),
            jax.ShapeDtypeStruct((N, 16), jnp.float32),
        ],
    )(adj, u1, b1.reshape(1, 32), W2)
    out = pl.pallas_call(
        _layer23_kernel,
        grid=(2, NI2),
        in_specs=[
            pl.BlockSpec((BM2, N), lambda l, i: (i, 0)),
            pl.BlockSpec((N, 16), lambda l, i: (0, 0)),
            pl.BlockSpec((1, 16), lambda l, i: (0, 0)),
            pl.BlockSpec((16, 1), lambda l, i: (0, 0)),
            pl.BlockSpec((1, 1), lambda l, i: (0, 0)),
            pl.BlockSpec((1, N), lambda l, i: (0, 0)),
            pl.BlockSpec((1, 1), lambda l, i: (0, 0)),
        ],
        out_specs=pl.BlockSpec((1, 1), lambda l, i: (0, 0)),
        out_shape=jax.ShapeDtypeStruct((1, 1), jnp.float32),
        scratch_shapes=[
            pltpu.VMEM((N, 128), jnp.float32),
        ],
    )(adj_bf, u2, b2.reshape(1, 16), W3, b3.reshape(1, 1), Wr,
      br.reshape(1, 1))
    return out.reshape(1)


# speedup vs baseline: 1.1841x; 1.0175x over previous
import jax
import jax.numpy as jnp
from jax.experimental import pallas as pl
from jax.experimental.pallas import tpu as pltpu

N = 8192
BM1 = 512
NI1 = N // BM1
BM2 = 1024
NI2 = N // BM2


def _layer1_kernel(adj_ref, x_ref, w1_ref, b1_ref, w2_ref, abf_ref, u2_ref,
                   u1_s):
    @pl.when(pl.program_id(0) == 0)
    def _proj():
        u1_s[...] = jnp.dot(x_ref[...], w1_ref[...],
                            preferred_element_type=jnp.float32
                            ).astype(jnp.bfloat16)
    a = adj_ref[...].astype(jnp.bfloat16)
    abf_ref[...] = a
    h = jnp.maximum(
        jnp.dot(a, u1_s[...], preferred_element_type=jnp.float32)
        + b1_ref[...], 0.0)
    u2_ref[...] = jnp.dot(h, w2_ref[...], preferred_element_type=jnp.float32)


def _layer23_kernel(abf_ref, u2_ref, b2_ref, w3_ref, b3_ref, wr_ref, br_ref,
                    o_ref, s_ref):
    l = pl.program_id(0)
    i = pl.program_id(1)
    a = abf_ref[...]
    row = pl.ds(i * BM2, BM2)

    @pl.when(l == 0)
    def _layer2():
        h = jnp.maximum(
            jnp.dot(a, u2_ref[...].astype(jnp.bfloat16),
                    preferred_element_type=jnp.float32)
            + b2_ref[...], 0.0)
        s_ref[row, 0:1] = jnp.dot(h, w3_ref[...],
                                  preferred_element_type=jnp.float32)

    @pl.when(l == 1)
    def _layer3():
        h = jnp.maximum(
            jnp.dot(a, s_ref[:, 0:1].astype(jnp.bfloat16),
                    preferred_element_type=jnp.float32)
            + b3_ref[...], 0.0)
        s_ref[row, 1:2] = h

    @pl.when(jnp.logical_and(l == 1, i == NI2 - 1))
    def _readout():
        o_ref[...] = br_ref[...] + jnp.sum(
            wr_ref[...] * s_ref[:, 1:2].reshape(1, N), keepdims=True)


def kernel(x, adj, W1, b1, W2, b2, W3, b3, Wr, br):
    adj_bf, u2 = pl.pallas_call(
        _layer1_kernel,
        grid=(NI1,),
        in_specs=[
            pl.BlockSpec((BM1, N), lambda i: (i, 0)),
            pl.BlockSpec((N, 128), lambda i: (0, 0)),
            pl.BlockSpec((128, 32), lambda i: (0, 0)),
            pl.BlockSpec((1, 32), lambda i: (0, 0)),
            pl.BlockSpec((32, 16), lambda i: (0, 0)),
        ],
        scratch_shapes=[pltpu.VMEM((N, 32), jnp.bfloat16)],
        out_specs=[
            pl.BlockSpec((BM1, N), lambda i: (i, 0)),
            pl.BlockSpec((BM1, 16), lambda i: (i, 0)),
        ],
        out_shape=[
            jax.ShapeDtypeStruct((N, N), jnp.bfloat16),
            jax.ShapeDtypeStruct((N, 16), jnp.float32),
        ],
    )(adj, x, W1, b1.reshape(1, 32), W2)
    out = pl.pallas_call(
        _layer23_kernel,
        grid=(2, NI2),
        in_specs=[
            pl.BlockSpec((BM2, N), lambda l, i: (i, 0)),
            pl.BlockSpec((N, 16), lambda l, i: (0, 0)),
            pl.BlockSpec((1, 16), lambda l, i: (0, 0)),
            pl.BlockSpec((16, 1), lambda l, i: (0, 0)),
            pl.BlockSpec((1, 1), lambda l, i: (0, 0)),
            pl.BlockSpec((1, N), lambda l, i: (0, 0)),
            pl.BlockSpec((1, 1), lambda l, i: (0, 0)),
        ],
        out_specs=pl.BlockSpec((1, 1), lambda l, i: (0, 0)),
        out_shape=jax.ShapeDtypeStruct((1, 1), jnp.float32),
        scratch_shapes=[
            pltpu.VMEM((N, 128), jnp.float32),
        ],
    )(adj_bf, u2, b2.reshape(1, 16), W3, b3.reshape(1, 1), Wr,
      br.reshape(1, 1))
    return out.reshape(1)


# speedup vs baseline: 1.1910x; 1.0058x over previous
import jax
import jax.numpy as jnp
from jax.experimental import pallas as pl
from jax.experimental.pallas import tpu as pltpu

N = 8192
BM1 = 512
NI1 = N // BM1
BM2 = 512
TAIL = 1024
NH = N - TAIL
NIH = NH // BM2


def _layer1_kernel(adj_ref, x_ref, w1_ref, b1_ref, w2_ref, abf_ref, u2_ref,
                   u1_s):
    @pl.when(pl.program_id(0) == 0)
    def _proj():
        u1_s[...] = jnp.dot(x_ref[...], w1_ref[...],
                            preferred_element_type=jnp.float32
                            ).astype(jnp.bfloat16)
    a = adj_ref[...].astype(jnp.bfloat16)
    abf_ref[...] = a
    h = jnp.maximum(
        jnp.dot(a, u1_s[...], preferred_element_type=jnp.float32)
        + b1_ref[...], 0.0)
    u2_ref[...] = jnp.dot(h, w2_ref[...], preferred_element_type=jnp.float32
                          ).astype(jnp.bfloat16)


def _layer23_kernel(abf_ref, atail_ref, u2_ref, b2_ref, w3_ref, b3_ref,
                    wr_ref, br_ref, o_ref, s_ref):
    l = pl.program_id(0)
    i = pl.program_id(1)
    row = pl.ds(i * BM2, BM2)

    @pl.when(l == 0)
    def _layer2():
        h = jnp.maximum(
            jnp.dot(abf_ref[...], u2_ref[...],
                    preferred_element_type=jnp.float32)
            + b2_ref[...], 0.0)
        s_ref[row, 0:1] = jnp.dot(h, w3_ref[...],
                                  preferred_element_type=jnp.float32)

    @pl.when(jnp.logical_and(l == 0, i == NIH - 1))
    def _layer2_tail():
        h = jnp.maximum(
            jnp.dot(atail_ref[...], u2_ref[...],
                    preferred_element_type=jnp.float32)
            + b2_ref[...], 0.0)
        s_ref[pl.ds(NH, TAIL), 0:1] = jnp.dot(
            h, w3_ref[...], preferred_element_type=jnp.float32)

    @pl.when(l == 1)
    def _layer3():
        v = s_ref[:, 0:1].astype(jnp.bfloat16)
        h = jnp.maximum(
            jnp.dot(abf_ref[...], v, preferred_element_type=jnp.float32)
            + b3_ref[...], 0.0)
        s_ref[row, 1:2] = h

    @pl.when(jnp.logical_and(l == 1, i == NIH - 1))
    def _layer3_tail_and_readout():
        v = s_ref[:, 0:1].astype(jnp.bfloat16)
        h = jnp.maximum(
            jnp.dot(atail_ref[...], v, preferred_element_type=jnp.float32)
            + b3_ref[...], 0.0)
        s_ref[pl.ds(NH, TAIL), 1:2] = h
        o_ref[...] = br_ref[...] + jnp.sum(
            wr_ref[...] * s_ref[:, 1:2].reshape(1, N), keepdims=True)


def kernel(x, adj, W1, b1, W2, b2, W3, b3, Wr, br):
    adj_bf, u2 = pl.pallas_call(
        _layer1_kernel,
        grid=(NI1,),
        in_specs=[
            pl.BlockSpec((BM1, N), lambda i: (i, 0)),
            pl.BlockSpec((N, 128), lambda i: (0, 0)),
            pl.BlockSpec((128, 32), lambda i: (0, 0)),
            pl.BlockSpec((1, 32), lambda i: (0, 0)),
            pl.BlockSpec((32, 16), lambda i: (0, 0)),
        ],
        scratch_shapes=[pltpu.VMEM((N, 32), jnp.bfloat16)],
        out_specs=[
            pl.BlockSpec((BM1, N), lambda i: (i, 0)),
            pl.BlockSpec((BM1, 16), lambda i: (i, 0)),
        ],
        out_shape=[
            jax.ShapeDtypeStruct((N, N), jnp.bfloat16),
            jax.ShapeDtypeStruct((N, 16), jnp.bfloat16),
        ],
    )(adj, x, W1, b1.reshape(1, 32), W2)
    out = pl.pallas_call(
        _layer23_kernel,
        grid=(2, NIH),
        in_specs=[
            pl.BlockSpec((BM2, N), lambda l, i: (i, 0)),
            pl.BlockSpec((TAIL, N), lambda l, i: (NH // TAIL, 0)),
            pl.BlockSpec((N, 16), lambda l, i: (0, 0)),
            pl.BlockSpec((1, 16), lambda l, i: (0, 0)),
            pl.BlockSpec((16, 1), lambda l, i: (0, 0)),
            pl.BlockSpec((1, 1), lambda l, i: (0, 0)),
            pl.BlockSpec((1, N), lambda l, i: (0, 0)),
            pl.BlockSpec((1, 1), lambda l, i: (0, 0)),
        ],
        out_specs=pl.BlockSpec((1, 1), lambda l, i: (0, 0)),
        out_shape=jax.ShapeDtypeStruct((1, 1), jnp.float32),
        scratch_shapes=[
            pltpu.VMEM((N, 128), jnp.float32),
        ],
    )(adj_bf, adj_bf, u2, b2.reshape(1, 16), W3, b3.reshape(1, 1), Wr,
      br.reshape(1, 1))
    return out.reshape(1)
